# baseline (speedup 1.0000x reference)
import jax
import jax.numpy as jnp
from jax.experimental import pallas as pl
from jax.experimental.pallas import tpu as pltpu

BATCH = 16384
DIM = 16
NSTEP = 64
PER = BATCH // NSTEP


def _tc_body(u_s, i_s, ut_hbm, it_hbm, out_ref, rows_u, rows_i, sem):
    step = pl.program_id(0)
    base = step * PER
    copies = []
    for l in range(PER):
        iu = u_s[base + l]
        ii = i_s[base + l]
        copies.append(pltpu.make_async_copy(
            ut_hbm.at[pl.ds(iu, 1)], rows_u.at[pl.ds(l, 1)], sem))
        copies[-1].start()
        copies.append(pltpu.make_async_copy(
            it_hbm.at[pl.ds(ii, 1)], rows_i.at[pl.ds(l, 1)], sem))
        copies[-1].start()
    for c in copies:
        c.wait()
    out_ref[...] = jnp.sum(rows_u[...] * rows_i[...], axis=1)


@jax.jit
def kernel(u, i, user_table, item_table):
    grid_spec = pltpu.PrefetchScalarGridSpec(
        num_scalar_prefetch=2,
        grid=(NSTEP,),
        in_specs=[
            pl.BlockSpec(memory_space=pltpu.HBM),
            pl.BlockSpec(memory_space=pltpu.HBM),
        ],
        out_specs=pl.BlockSpec((PER,), lambda s, u_s, i_s: (s,)),
        scratch_shapes=[
            pltpu.VMEM((PER, DIM), jnp.float32),
            pltpu.VMEM((PER, DIM), jnp.float32),
            pltpu.SemaphoreType.DMA,
        ],
    )
    f = pl.pallas_call(
        _tc_body,
        grid_spec=grid_spec,
        out_shape=jax.ShapeDtypeStruct((BATCH,), jnp.float32),
    )
    return f(u.astype(jnp.int32), i.astype(jnp.int32),
             user_table, item_table)


# speedup vs baseline: 1.1300x; 1.1300x over previous
import jax
import jax.numpy as jnp
from jax import lax
from jax.experimental import pallas as pl
from jax.experimental.pallas import tpu as pltpu
from jax.experimental.pallas import tpu_sc as plsc

NC = 2
NS = 16
NW = NC * NS
L = 16
BATCH = 16384
DIM = 16
HALF = BATCH // 2

BPW = HALF // NW
NCHUNK = 2
CHUNK = BPW // NCHUNK

NSTEP = 32
PER = HALF // NSTEP


def _sc_body(u_hbm, i_hbm, ut_hbm, it_hbm, out_hbm,
             idx_u, idx_i, ue, ie, out_v, *sems):
    wid = lax.axis_index("s") * NC + lax.axis_index("c")
    base = wid * BPW

    pltpu.sync_copy(u_hbm.at[wid], idx_u)
    pltpu.sync_copy(i_hbm.at[wid], idx_i)

    def fire_chunk(j, buf):
        def enq(g, carry):
            r0 = g * L
            iu_vec = idx_u[pl.ds(j * CHUNK + r0, L)]
            ii_vec = idx_i[pl.ds(j * CHUNK + r0, L)]
            for l in range(L):
                pltpu.async_copy(ut_hbm.at[iu_vec[l]],
                                 ue.at[buf, r0 + l], sems[0])
                pltpu.async_copy(it_hbm.at[ii_vec[l]],
                                 ie.at[buf, r0 + l], sems[1])
            return carry

        lax.fori_loop(0, CHUNK // L, enq, 0)

    def drain_chunk(buf):
        pltpu.make_async_copy(ut_hbm.at[pl.ds(0, CHUNK)],
                              ue.at[buf], sems[0]).wait()
        pltpu.make_async_copy(it_hbm.at[pl.ds(0, CHUNK)],
                              ie.at[buf], sems[1]).wait()

    def compute_chunk(j, buf):
        def group(g, carry):
            r0 = g * L
            rows = lax.iota(jnp.int32, L) + r0
            acc = jnp.zeros((L,), jnp.float32)
            for d in range(DIM):
                col = jnp.full((L,), d, jnp.int32)
                uc = plsc.load_gather(ue.at[buf], [rows, col])
                ic = plsc.load_gather(ie.at[buf], [rows, col])
                acc = acc + uc * ic
            out_v[pl.ds(j * CHUNK + r0, L)] = acc
            return carry

        lax.fori_loop(0, CHUNK // L, group, 0)

    fire_chunk(0, 0)
    for j in range(NCHUNK):
        drain_chunk(j % 2)
        if j + 1 < NCHUNK:
            fire_chunk(j + 1, (j + 1) % 2)
        compute_chunk(j, j % 2)

    pltpu.sync_copy(out_v, out_hbm.at[pl.ds(base, BPW)])


def _tc_body(u_s, i_s, ut_hbm, it_hbm, out_ref, rows_u, rows_i, sem):
    step = pl.program_id(0)
    base = step * PER
    copies = []
    for l in range(PER):
        iu = u_s[base + l]
        ii = i_s[base + l]
        copies.append(pltpu.make_async_copy(
            ut_hbm.at[pl.ds(iu, 1)], rows_u.at[pl.ds(l, 1)], sem))
        copies[-1].start()
        copies.append(pltpu.make_async_copy(
            it_hbm.at[pl.ds(ii, 1)], rows_i.at[pl.ds(l, 1)], sem))
        copies[-1].start()
    for c in copies:
        c.wait()
    out_ref[...] = jnp.sum(rows_u[...] * rows_i[...], axis=1)


@jax.jit
def kernel(u, i, user_table, item_table):
    u32 = u.astype(jnp.int32)
    i32 = i.astype(jnp.int32)

    u2 = u32[:HALF].reshape(NW, BPW)
    i2 = i32[:HALF].reshape(NW, BPW)
    mesh = plsc.VectorSubcoreMesh(core_axis_name="c", subcore_axis_name="s")
    sc = pl.kernel(
        _sc_body,
        out_type=jax.ShapeDtypeStruct((HALF,), jnp.float32),
        mesh=mesh,
        compiler_params=pltpu.CompilerParams(needs_layout_passes=False),
        scratch_types=[
            pltpu.VMEM((BPW,), jnp.int32),
            pltpu.VMEM((BPW,), jnp.int32),
            pltpu.VMEM((2, CHUNK, DIM), jnp.float32),
            pltpu.VMEM((2, CHUNK, DIM), jnp.float32),
            pltpu.VMEM((BPW,), jnp.float32),
        ] + [pltpu.SemaphoreType.DMA] * 2,
    )
    out_sc = sc(u2, i2, user_table, item_table)

    grid_spec = pltpu.PrefetchScalarGridSpec(
        num_scalar_prefetch=2,
        grid=(NSTEP,),
        in_specs=[
            pl.BlockSpec(memory_space=pltpu.HBM),
            pl.BlockSpec(memory_space=pltpu.HBM),
        ],
        out_specs=pl.BlockSpec((PER,), lambda s, u_s, i_s: (s,)),
        scratch_shapes=[
            pltpu.VMEM((PER, DIM), jnp.float32),
            pltpu.VMEM((PER, DIM), jnp.float32),
            pltpu.SemaphoreType.DMA,
        ],
    )
    tc = pl.pallas_call(
        _tc_body,
        grid_spec=grid_spec,
        out_shape=jax.ShapeDtypeStruct((HALF,), jnp.float32),
    )
    out_tc = tc(u32[HALF:], i32[HALF:], user_table, item_table)

    return jnp.concatenate([out_sc, out_tc])


# speedup vs baseline: 1.3105x; 1.1597x over previous
import jax
import jax.numpy as jnp
from jax import lax
from jax.experimental import pallas as pl
from jax.experimental.pallas import tpu as pltpu
from jax.experimental.pallas import tpu_sc as plsc

NC = 2
NS = 16
NW = NC * NS
L = 16
BATCH = 16384
DIM = 16
BPW = BATCH // NW
NCHUNK = 4
CHUNK = BPW // NCHUNK


def _body(u_hbm, i_hbm, ut_hbm, it_hbm, out_hbm,
          idx_u, idx_i, ue, ie, out_v, *sems):
    wid = lax.axis_index("s") * NC + lax.axis_index("c")
    base = wid * BPW

    pltpu.sync_copy(u_hbm.at[wid], idx_u)
    pltpu.sync_copy(i_hbm.at[wid], idx_i)

    def fire_chunk(j, buf):
        def enq(g, carry):
            r0 = g * L
            iu_vec = idx_u[pl.ds(j * CHUNK + r0, L)]
            ii_vec = idx_i[pl.ds(j * CHUNK + r0, L)]
            for l in range(L):
                pltpu.async_copy(ut_hbm.at[iu_vec[l]],
                                 ue.at[buf, r0 + l], sems[0])
                pltpu.async_copy(it_hbm.at[ii_vec[l]],
                                 ie.at[buf, r0 + l], sems[1])
            return carry

        lax.fori_loop(0, CHUNK // L, enq, 0)

    def drain_chunk(buf):
        pltpu.make_async_copy(ut_hbm.at[pl.ds(0, CHUNK)],
                              ue.at[buf], sems[0]).wait()
        pltpu.make_async_copy(it_hbm.at[pl.ds(0, CHUNK)],
                              ie.at[buf], sems[1]).wait()

    def compute_chunk(j, buf):
        def group(g, carry):
            r0 = g * L
            rows = lax.iota(jnp.int32, L) + r0
            acc = jnp.zeros((L,), jnp.float32)
            for d in range(DIM):
                col = jnp.full((L,), d, jnp.int32)
                uc = plsc.load_gather(ue.at[buf], [rows, col])
                ic = plsc.load_gather(ie.at[buf], [rows, col])
                acc = acc + uc * ic
            out_v[pl.ds(j * CHUNK + r0, L)] = acc
            return carry

        lax.fori_loop(0, CHUNK // L, group, 0)

    fire_chunk(0, 0)
    for j in range(NCHUNK):
        drain_chunk(j % 2)
        if j + 1 < NCHUNK:
            fire_chunk(j + 1, (j + 1) % 2)
        compute_chunk(j, j % 2)

    pltpu.sync_copy(out_v, out_hbm.at[pl.ds(base, BPW)])


@jax.jit
def kernel(u, i, user_table, item_table):
    u2 = u.astype(jnp.int32).reshape(NW, BPW)
    i2 = i.astype(jnp.int32).reshape(NW, BPW)
    mesh = plsc.VectorSubcoreMesh(core_axis_name="c", subcore_axis_name="s")
    f = pl.kernel(
        _body,
        out_type=jax.ShapeDtypeStruct((BATCH,), jnp.float32),
        mesh=mesh,
        compiler_params=pltpu.CompilerParams(needs_layout_passes=False),
        scratch_types=[
            pltpu.VMEM((BPW,), jnp.int32),
            pltpu.VMEM((BPW,), jnp.int32),
            pltpu.VMEM((2, CHUNK, DIM), jnp.float32),
            pltpu.VMEM((2, CHUNK, DIM), jnp.float32),
            pltpu.VMEM((BPW,), jnp.float32),
        ] + [pltpu.SemaphoreType.DMA] * 2,
    )
    return f(u2, i2, user_table, item_table)
